# baseline (speedup 1.0000x reference)
import functools

import jax
import jax.numpy as jnp
from jax.experimental import pallas as pl
from jax.experimental.pallas import tpu as pltpu

B, K, T, OUT = 64, 64, 256, 256
N = B * K
ROWS = 128
GRID = N // ROWS


def _gcn_kernel(posT_ref, x_ref,
                w1_ref, b1_ref, w2_ref, b2_ref, w3_ref, b3_ref, w4_ref, b4_ref,
                out_ref):
    p = posT_ref[...]

    d2 = jnp.zeros((ROWS, ROWS), jnp.float32)
    for c in range(3):
        row = p[c:c + 1, :]
        col = row.reshape(ROWS, 1)
        d2 = d2 + (col - row) ** 2

    ri = jax.lax.broadcasted_iota(jnp.int32, (ROWS, ROWS), 0)
    ci = jax.lax.broadcasted_iota(jnp.int32, (ROWS, ROWS), 1)
    same_batch = (ri // K) == (ci // K)
    diag = ri == ci

    w = jnp.where(same_batch & (~diag),
                  1.0 / (jnp.sqrt(d2) + 1e-6),
                  0.0)
    w = w + jnp.where(diag, 1.0, 0.0)

    deg = jnp.sum(w, axis=1, keepdims=True)
    dis = jax.lax.rsqrt(deg)
    m = dis * w * dis.reshape(1, ROWS)

    x = x_ref[...]
    for wref, bref, act in ((w1_ref, b1_ref, True),
                            (w2_ref, b2_ref, True),
                            (w3_ref, b3_ref, True),
                            (w4_ref, b4_ref, False)):
        xw = jnp.dot(x, wref[...], preferred_element_type=jnp.float32)
        y = jnp.dot(m, xw, preferred_element_type=jnp.float32) + bref[...]
        x = jnp.where(y > 0, y, 0.01 * y) if act else y

    out_ref[...] = x


@jax.jit
def kernel(feat, pos, W1, b1, W2, b2, W3, b3, W4, b4):
    x = feat.reshape(N, T)
    posT = pos.reshape(N, 3).T
    row_spec = pl.BlockSpec((ROWS, T), lambda i: (i, 0))
    full = lambda shape: pl.BlockSpec(shape, lambda i: (0, 0))

    out = pl.pallas_call(
        _gcn_kernel,
        grid=(GRID,),
        in_specs=[
            pl.BlockSpec((3, ROWS), lambda i: (0, i)),
            row_spec,
            full((T, T)), full((1, T)),
            full((T, T)), full((1, T)),
            full((T, T)), full((1, T)),
            full((T, OUT)), full((1, OUT)),
        ],
        out_specs=pl.BlockSpec((ROWS, OUT), lambda i: (i, 0)),
        out_shape=jax.ShapeDtypeStruct((N, OUT), jnp.float32),
        compiler_params=pltpu.CompilerParams(
            dimension_semantics=("parallel",),
        ),
    )(posT, x,
      W1, b1.reshape(1, T), W2, b2.reshape(1, T),
      W3, b3.reshape(1, T), W4, b4.reshape(1, OUT))

    return out.reshape(B, K, OUT)


# speedup vs baseline: 1.7180x; 1.7180x over previous
import jax
import jax.numpy as jnp
from jax.experimental import pallas as pl
from jax.experimental.pallas import tpu as pltpu

B, K, T, OUT = 64, 64, 256, 256
N = B * K
ROWS = 256
MBLK = 128
GRID = N // ROWS
NSUB = ROWS // MBLK


def _build_m(p):
    d2 = jnp.zeros((MBLK, MBLK), jnp.float32)
    for c in range(3):
        row = p[c:c + 1, :]
        col = row.reshape(MBLK, 1)
        d2 = d2 + (col - row) ** 2

    ri = jax.lax.broadcasted_iota(jnp.int32, (MBLK, MBLK), 0)
    ci = jax.lax.broadcasted_iota(jnp.int32, (MBLK, MBLK), 1)
    same_batch = (ri // K) == (ci // K)
    diag = ri == ci

    w = jnp.where(same_batch & (~diag),
                  1.0 / (jnp.sqrt(d2) + 1e-6),
                  0.0)
    w = w + jnp.where(diag, 1.0, 0.0)

    deg = jnp.sum(w, axis=1, keepdims=True)
    dis = jax.lax.rsqrt(deg)
    return dis * w * dis.reshape(1, MBLK)


def _gcn_kernel(posT_ref, x_ref,
                w1_ref, b1_ref, w2_ref, b2_ref, w3_ref, b3_ref, w4_ref, b4_ref,
                out_ref):
    p = posT_ref[...]
    ms = [_build_m(p[:, j * MBLK:(j + 1) * MBLK]) for j in range(NSUB)]

    x = x_ref[...]
    for wref, bref, act in ((w1_ref, b1_ref, True),
                            (w2_ref, b2_ref, True),
                            (w3_ref, b3_ref, True),
                            (w4_ref, b4_ref, False)):
        xw = jnp.dot(x, wref[...], preferred_element_type=jnp.float32)
        ys = [jnp.dot(ms[j], xw[j * MBLK:(j + 1) * MBLK, :],
                      preferred_element_type=jnp.float32)
              for j in range(NSUB)]
        y = (jnp.concatenate(ys, axis=0) if NSUB > 1 else ys[0]) + bref[...]
        x = jnp.where(y > 0, y, 0.01 * y) if act else y

    out_ref[...] = x


@jax.jit
def kernel(feat, pos, W1, b1, W2, b2, W3, b3, W4, b4):
    x = feat.reshape(N, T)
    posT = pos.reshape(N, 3).T
    row_spec = pl.BlockSpec((ROWS, T), lambda i: (i, 0))
    full = lambda shape: pl.BlockSpec(shape, lambda i: (0, 0))

    out = pl.pallas_call(
        _gcn_kernel,
        grid=(GRID,),
        in_specs=[
            pl.BlockSpec((3, ROWS), lambda i: (0, i)),
            row_spec,
            full((T, T)), full((1, T)),
            full((T, T)), full((1, T)),
            full((T, T)), full((1, T)),
            full((T, OUT)), full((1, OUT)),
        ],
        out_specs=pl.BlockSpec((ROWS, OUT), lambda i: (i, 0)),
        out_shape=jax.ShapeDtypeStruct((N, OUT), jnp.float32),
        compiler_params=pltpu.CompilerParams(
            dimension_semantics=("parallel",),
        ),
    )(posT, x,
      W1, b1.reshape(1, T), W2, b2.reshape(1, T),
      W3, b3.reshape(1, T), W4, b4.reshape(1, OUT))

    return out.reshape(B, K, OUT)


# speedup vs baseline: 2.5847x; 1.5045x over previous
import jax
import jax.numpy as jnp
from jax.experimental import pallas as pl
from jax.experimental.pallas import tpu as pltpu

B, K, T, OUT = 64, 64, 256, 256
N = B * K
ROWS = 512
MBLK = 128
GRID = N // ROWS
NSUB = ROWS // MBLK


def _build_m(p):
    d2 = jnp.zeros((MBLK, MBLK), jnp.float32)
    for c in range(3):
        row = p[c:c + 1, :]
        col = row.reshape(MBLK, 1)
        d2 = d2 + (col - row) ** 2

    ri = jax.lax.broadcasted_iota(jnp.int32, (MBLK, MBLK), 0)
    ci = jax.lax.broadcasted_iota(jnp.int32, (MBLK, MBLK), 1)
    same_batch = (ri // K) == (ci // K)
    diag = ri == ci

    w = jnp.where(same_batch & (~diag),
                  1.0 / (jnp.sqrt(d2) + 1e-6),
                  0.0)
    w = w + jnp.where(diag, 1.0, 0.0)

    deg = jnp.sum(w, axis=1, keepdims=True)
    dis = jax.lax.rsqrt(deg)
    return dis * w * dis.reshape(1, MBLK)


def _gcn_kernel(posT_ref, x_ref,
                w1_ref, b1_ref, w2_ref, b2_ref, w3_ref, b3_ref, w4_ref, b4_ref,
                out_ref):
    p = posT_ref[...]
    ms = [_build_m(p[:, j * MBLK:(j + 1) * MBLK]) for j in range(NSUB)]

    x = x_ref[...]
    for wref, bref, act in ((w1_ref, b1_ref, True),
                            (w2_ref, b2_ref, True),
                            (w3_ref, b3_ref, True),
                            (w4_ref, b4_ref, False)):
        xw = jnp.dot(x, wref[...], preferred_element_type=jnp.float32)
        ys = [jnp.dot(ms[j], xw[j * MBLK:(j + 1) * MBLK, :],
                      preferred_element_type=jnp.float32)
              for j in range(NSUB)]
        y = (jnp.concatenate(ys, axis=0) if NSUB > 1 else ys[0]) + bref[...]
        x = jnp.where(y > 0, y, 0.01 * y) if act else y

    out_ref[...] = x


@jax.jit
def kernel(feat, pos, W1, b1, W2, b2, W3, b3, W4, b4):
    x = feat.reshape(N, T)
    posT = pos.reshape(N, 3).T
    row_spec = pl.BlockSpec((ROWS, T), lambda i: (i, 0))
    full = lambda shape: pl.BlockSpec(shape, lambda i: (0, 0))

    out = pl.pallas_call(
        _gcn_kernel,
        grid=(GRID,),
        in_specs=[
            pl.BlockSpec((3, ROWS), lambda i: (0, i)),
            row_spec,
            full((T, T)), full((1, T)),
            full((T, T)), full((1, T)),
            full((T, T)), full((1, T)),
            full((T, OUT)), full((1, OUT)),
        ],
        out_specs=pl.BlockSpec((ROWS, OUT), lambda i: (i, 0)),
        out_shape=jax.ShapeDtypeStruct((N, OUT), jnp.float32),
        compiler_params=pltpu.CompilerParams(
            dimension_semantics=("parallel",),
        ),
    )(posT, x,
      W1, b1.reshape(1, T), W2, b2.reshape(1, T),
      W3, b3.reshape(1, T), W4, b4.reshape(1, OUT))

    return out.reshape(B, K, OUT)


# speedup vs baseline: 3.3784x; 1.3071x over previous
import jax
import jax.numpy as jnp
from jax.experimental import pallas as pl
from jax.experimental.pallas import tpu as pltpu

B, K, T, OUT = 64, 64, 256, 256
N = B * K
ROWS = 1024
MBLK = 128
GRID = N // ROWS
NSUB = ROWS // MBLK


def _build_m(p):
    d2 = jnp.zeros((MBLK, MBLK), jnp.float32)
    for c in range(3):
        row = p[c:c + 1, :]
        col = row.reshape(MBLK, 1)
        d2 = d2 + (col - row) ** 2

    ri = jax.lax.broadcasted_iota(jnp.int32, (MBLK, MBLK), 0)
    ci = jax.lax.broadcasted_iota(jnp.int32, (MBLK, MBLK), 1)
    same_batch = (ri // K) == (ci // K)
    diag = ri == ci

    w = jnp.where(same_batch & (~diag),
                  1.0 / (jnp.sqrt(d2) + 1e-6),
                  0.0)
    w = w + jnp.where(diag, 1.0, 0.0)

    deg = jnp.sum(w, axis=1, keepdims=True)
    dis = jax.lax.rsqrt(deg)
    return dis * w * dis.reshape(1, MBLK)


def _gcn_kernel(posT_ref, x_ref,
                w1_ref, b1_ref, w2_ref, b2_ref, w3_ref, b3_ref, w4_ref, b4_ref,
                out_ref):
    p = posT_ref[...]
    ms = [_build_m(p[:, j * MBLK:(j + 1) * MBLK]) for j in range(NSUB)]

    x = x_ref[...]
    for wref, bref, act in ((w1_ref, b1_ref, True),
                            (w2_ref, b2_ref, True),
                            (w3_ref, b3_ref, True),
                            (w4_ref, b4_ref, False)):
        xw = jnp.dot(x, wref[...], preferred_element_type=jnp.float32)
        ys = [jnp.dot(ms[j], xw[j * MBLK:(j + 1) * MBLK, :],
                      preferred_element_type=jnp.float32)
              for j in range(NSUB)]
        y = (jnp.concatenate(ys, axis=0) if NSUB > 1 else ys[0]) + bref[...]
        x = jnp.where(y > 0, y, 0.01 * y) if act else y

    out_ref[...] = x


@jax.jit
def kernel(feat, pos, W1, b1, W2, b2, W3, b3, W4, b4):
    x = feat.reshape(N, T)
    posT = pos.reshape(N, 3).T
    row_spec = pl.BlockSpec((ROWS, T), lambda i: (i, 0))
    full = lambda shape: pl.BlockSpec(shape, lambda i: (0, 0))

    out = pl.pallas_call(
        _gcn_kernel,
        grid=(GRID,),
        in_specs=[
            pl.BlockSpec((3, ROWS), lambda i: (0, i)),
            row_spec,
            full((T, T)), full((1, T)),
            full((T, T)), full((1, T)),
            full((T, T)), full((1, T)),
            full((T, OUT)), full((1, OUT)),
        ],
        out_specs=pl.BlockSpec((ROWS, OUT), lambda i: (i, 0)),
        out_shape=jax.ShapeDtypeStruct((N, OUT), jnp.float32),
        compiler_params=pltpu.CompilerParams(
            dimension_semantics=("parallel",),
        ),
    )(posT, x,
      W1, b1.reshape(1, T), W2, b2.reshape(1, T),
      W3, b3.reshape(1, T), W4, b4.reshape(1, OUT))

    return out.reshape(B, K, OUT)


# speedup vs baseline: 3.4807x; 1.0303x over previous
import jax
import jax.numpy as jnp
from jax.experimental import pallas as pl
from jax.experimental.pallas import tpu as pltpu

B, K, T, OUT = 64, 64, 256, 256
N = B * K
ROWS = 2048
MBLK = 128
GRID = N // ROWS
NSUB = ROWS // MBLK


def _build_m(p):
    d2 = jnp.zeros((MBLK, MBLK), jnp.float32)
    for c in range(3):
        row = p[c:c + 1, :]
        col = row.reshape(MBLK, 1)
        d2 = d2 + (col - row) ** 2

    ri = jax.lax.broadcasted_iota(jnp.int32, (MBLK, MBLK), 0)
    ci = jax.lax.broadcasted_iota(jnp.int32, (MBLK, MBLK), 1)
    same_batch = (ri // K) == (ci // K)
    diag = ri == ci

    w = jnp.where(same_batch & (~diag),
                  1.0 / (jnp.sqrt(d2) + 1e-6),
                  0.0)
    w = w + jnp.where(diag, 1.0, 0.0)

    deg = jnp.sum(w, axis=1, keepdims=True)
    dis = jax.lax.rsqrt(deg)
    return dis * w * dis.reshape(1, MBLK)


def _gcn_kernel(posT_ref, x_ref,
                w1_ref, b1_ref, w2_ref, b2_ref, w3_ref, b3_ref, w4_ref, b4_ref,
                out_ref):
    p = posT_ref[...]
    ms = [_build_m(p[:, j * MBLK:(j + 1) * MBLK]) for j in range(NSUB)]

    x = x_ref[...]
    for wref, bref, act in ((w1_ref, b1_ref, True),
                            (w2_ref, b2_ref, True),
                            (w3_ref, b3_ref, True),
                            (w4_ref, b4_ref, False)):
        xw = jnp.dot(x, wref[...], preferred_element_type=jnp.float32)
        ys = [jnp.dot(ms[j], xw[j * MBLK:(j + 1) * MBLK, :],
                      preferred_element_type=jnp.float32)
              for j in range(NSUB)]
        y = (jnp.concatenate(ys, axis=0) if NSUB > 1 else ys[0]) + bref[...]
        x = jnp.where(y > 0, y, 0.01 * y) if act else y

    out_ref[...] = x


@jax.jit
def kernel(feat, pos, W1, b1, W2, b2, W3, b3, W4, b4):
    x = feat.reshape(N, T)
    posT = pos.reshape(N, 3).T
    row_spec = pl.BlockSpec((ROWS, T), lambda i: (i, 0))
    full = lambda shape: pl.BlockSpec(shape, lambda i: (0, 0))

    out = pl.pallas_call(
        _gcn_kernel,
        grid=(GRID,),
        in_specs=[
            pl.BlockSpec((3, ROWS), lambda i: (0, i)),
            row_spec,
            full((T, T)), full((1, T)),
            full((T, T)), full((1, T)),
            full((T, T)), full((1, T)),
            full((T, OUT)), full((1, OUT)),
        ],
        out_specs=pl.BlockSpec((ROWS, OUT), lambda i: (i, 0)),
        out_shape=jax.ShapeDtypeStruct((N, OUT), jnp.float32),
        compiler_params=pltpu.CompilerParams(
            dimension_semantics=("parallel",),
        ),
    )(posT, x,
      W1, b1.reshape(1, T), W2, b2.reshape(1, T),
      W3, b3.reshape(1, T), W4, b4.reshape(1, OUT))

    return out.reshape(B, K, OUT)
